# Initial kernel scaffold; baseline (speedup 1.0000x reference)
import jax
import jax.numpy as jnp
from jax.experimental import pallas as pl


def kernel(fake, real):
    raise NotImplementedError("write your pallas kernel here")


# speedup vs baseline: 38.2328x; 38.2328x over previous
import functools

import jax
import jax.numpy as jnp
from jax import lax
from jax.experimental import pallas as pl
from jax.experimental.pallas import tpu as pltpu
from jax.experimental.pallas import tpu_sc as plsc

N = 16 * 3 * 512 * 512
NC = 2
NS = 16
NW = NC * NS
PER_W = N // NW
CHUNK = 16384
NCHUNK = PER_W // CHUNK
BINS = 256


def _hist_body(fake_hbm, real_hbm, out_hbm, buf, hist):
    wid = lax.axis_index("s") * NC + lax.axis_index("c")
    base = wid * PER_W

    zeros_f = jnp.zeros((16,), jnp.float32)
    ones_f = jnp.ones((16,), jnp.float32)

    for i in range(2 * BINS // 16):
        hist[pl.ds(i * 16, 16)] = zeros_f

    def accumulate(src_hbm, bin_off):
        off_vec = jnp.full((16,), bin_off, jnp.int32)

        def chunk_body(c, _):
            off = pl.multiple_of(base + c * CHUNK, CHUNK)
            pltpu.sync_copy(src_hbm.at[pl.ds(off, CHUNK)], buf)

            def body(i, _):
                x = buf[pl.ds(i * 16, 16)]
                t = (x * 128.0 + 128.0).astype(jnp.int32)
                idx = jnp.clip(t, 0, BINS - 1) + off_vec
                valid = (x >= -1.0) & (x <= 1.0)
                plsc.addupdate_scatter(hist, [idx], ones_f, mask=valid)
                return 0

            lax.fori_loop(0, CHUNK // 16, body, 0, unroll=4)
            return 0

        lax.fori_loop(0, NCHUNK, chunk_body, 0)

    accumulate(fake_hbm, 0)
    accumulate(real_hbm, BINS)
    pltpu.sync_copy(hist, out_hbm.at[wid])


def _sc_partial_hists(fake_flat, real_flat):
    mesh = plsc.VectorSubcoreMesh(core_axis_name="c", subcore_axis_name="s")
    kern = functools.partial(
        pl.kernel,
        out_type=jax.ShapeDtypeStruct((NW, 2 * BINS), jnp.float32),
        mesh=mesh,
        scratch_types=[
            pltpu.VMEM((CHUNK,), jnp.float32),
            pltpu.VMEM((2 * BINS,), jnp.float32),
        ],
        compiler_params=pltpu.CompilerParams(needs_layout_passes=False),
    )(_hist_body)
    return kern(fake_flat, real_flat)


def _loss_body(p_ref, o_ref):
    p = p_ref[...]
    tot = jnp.sum(p, axis=0, keepdims=True)
    hf = tot[:, :BINS]
    hr = tot[:, BINS:]
    sf = jnp.sum(hf)
    sr = jnp.sum(hr)
    d = hf / sf - hr / sr
    o_ref[...] = jnp.mean(d * d).reshape(1, 1)


def _tc_loss(partials):
    return pl.pallas_call(
        _loss_body,
        out_shape=jax.ShapeDtypeStruct((1, 1), jnp.float32),
    )(partials)


def kernel(fake, real):
    f = fake.reshape(-1)
    r = real.reshape(-1)
    partials = _sc_partial_hists(f, r)
    loss = _tc_loss(partials)
    return loss[0, 0]


# speedup vs baseline: 154.0037x; 4.0281x over previous
import functools

import jax
import jax.numpy as jnp
from jax import lax
from jax.experimental import pallas as pl
from jax.experimental.pallas import tpu as pltpu
from jax.experimental.pallas import tpu_sc as plsc

N = 16 * 3 * 512 * 512
NC = 2
NS = 16
NW = NC * NS
PER_W = N // NW
CHUNK = 16384
NCHUNK = PER_W // CHUNK
NPAIR = NCHUNK // 2
BINS = 256
K = 4


def _hist_body(fake_hbm, real_hbm, out_hbm, buf_a, buf_b, hist, sem_a, sem_b):
    wid = lax.axis_index("s") * NC + lax.axis_index("c")
    base = wid * PER_W

    zeros_f = jnp.zeros((16,), jnp.float32)
    ones_f = jnp.ones((16,), jnp.float32)

    for i in range(2 * K * BINS // 16):
        hist[pl.ds(i * 16, 16)] = zeros_f

    def process(buf, array_sel):
        @plsc.parallel_loop(0, CHUNK // 16, step=K)
        def _(i):
            for k in range(K):
                x = buf[pl.ds((i + k) * 16, 16)]
                t = (x * 128.0 + 128.0).astype(jnp.int32)
                idx = jnp.minimum(t, BINS - 1)
                valid = jnp.abs(x) <= 1.0
                region = hist.at[pl.ds((array_sel * K + k) * BINS, BINS)]
                plsc.addupdate_scatter(region, [idx], ones_f, mask=valid)

    def accumulate(src_hbm, array_sel):
        def copy_in(c, buf, sem):
            off = pl.multiple_of(base + c * CHUNK, CHUNK)
            return pltpu.async_copy(src_hbm.at[pl.ds(off, CHUNK)], buf, sem)

        copy_in(0, buf_a, sem_a)

        def pair_body(p, _):
            c0 = 2 * p
            pltpu.make_async_copy(src_hbm.at[pl.ds(0, CHUNK)], buf_a,
                                  sem_a).wait()
            copy_in(c0 + 1, buf_b, sem_b)
            process(buf_a, array_sel)
            pltpu.make_async_copy(src_hbm.at[pl.ds(0, CHUNK)], buf_b,
                                  sem_b).wait()

            @pl.when(p < NPAIR - 1)
            def _():
                copy_in(c0 + 2, buf_a, sem_a)

            process(buf_b, array_sel)
            return 0

        lax.fori_loop(0, NPAIR, pair_body, 0)

    accumulate(fake_hbm, 0)
    accumulate(real_hbm, 1)

    for a in range(2):
        for i in range(BINS // 16):
            acc = hist[pl.ds(a * K * BINS + i * 16, 16)]
            for k in range(1, K):
                acc = acc + hist[pl.ds((a * K + k) * BINS + i * 16, 16)]
            hist[pl.ds(a * K * BINS + i * 16, 16)] = acc
        pltpu.sync_copy(hist.at[pl.ds(a * K * BINS, BINS)],
                        out_hbm.at[wid, pl.ds(a * BINS, BINS)])


def _sc_partial_hists(fake_flat, real_flat):
    mesh = plsc.VectorSubcoreMesh(core_axis_name="c", subcore_axis_name="s")
    kern = functools.partial(
        pl.kernel,
        out_type=jax.ShapeDtypeStruct((NW, 2 * BINS), jnp.float32),
        mesh=mesh,
        scratch_types=[
            pltpu.VMEM((CHUNK,), jnp.float32),
            pltpu.VMEM((CHUNK,), jnp.float32),
            pltpu.VMEM((2 * K * BINS,), jnp.float32),
            pltpu.SemaphoreType.DMA,
            pltpu.SemaphoreType.DMA,
        ],
        compiler_params=pltpu.CompilerParams(needs_layout_passes=False),
    )(_hist_body)
    return kern(fake_flat, real_flat)


def _loss_body(p_ref, o_ref):
    p = p_ref[...]
    tot = jnp.sum(p, axis=0, keepdims=True)
    hf = tot[:, :BINS]
    hr = tot[:, BINS:]
    sf = jnp.sum(hf)
    sr = jnp.sum(hr)
    d = hf / sf - hr / sr
    o_ref[...] = jnp.mean(d * d).reshape(1, 1)


def _tc_loss(partials):
    return pl.pallas_call(
        _loss_body,
        out_shape=jax.ShapeDtypeStruct((1, 1), jnp.float32),
    )(partials)


def kernel(fake, real):
    f = fake.reshape(-1)
    r = real.reshape(-1)
    partials = _sc_partial_hists(f, r)
    loss = _tc_loss(partials)
    return loss[0, 0]
